# baseline (speedup 1.0000x reference)
import numpy as np

import jax
import jax.numpy as jnp
from jax.experimental import pallas as pl
from jax.experimental.pallas import tpu as pltpu

D = 128
L = 4
C_IN = 22


def _build_static():
    a = np.zeros((729, 3), dtype=np.int32)
    for cell in range(81):
        r = cell // 9
        c = cell % 9
        b = (r // 3) * 3 + c // 3
        for d in range(9):
            ci = cell * 9 + d
            a[ci, 0] = r * 9 + d
            a[ci, 1] = 81 + c * 9 + d
            a[ci, 2] = 162 + b * 9 + d
    A = np.zeros((243, 729), dtype=np.float32)
    for ci in range(729):
        for k in range(3):
            A[a[ci, k], ci] = 1.0
    R = np.zeros((729, 81), dtype=np.float32)
    for cell in range(81):
        for d in range(9):
            R[cell * 9 + d, cell] = 1.0
    return A / 9.0, A.T.copy() / 3.0, R


_A9_NP, _AT3_NP, _R_NP = _build_static()


def _mm(a, b):
    return jax.lax.dot_general(a, b, (((1,), (0,)), ((), ())),
                               preferred_element_type=jnp.float32,
                               precision=jax.lax.Precision.HIGHEST)


def _mm_nt(a, b):
    return jax.lax.dot_general(a, b, (((1,), (1,)), ((), ())),
                               preferred_element_type=jnp.float32,
                               precision=jax.lax.Precision.HIGHEST)


def _ln(x, g, b):
    m = jnp.mean(x, axis=-1, keepdims=True)
    v = jnp.mean((x - m) ** 2, axis=-1, keepdims=True)
    return (x - m) / jnp.sqrt(v + 1e-5) * g + b


def _fwd_kernel(x_ref, w1T, b1, w2T, b2, Rm, temb, semb, A9, AT3,
                vcT, vcB, vsT, vsB, nsG, nsB, ncG, ncB,
                ff1T, ff1B, ff2T, ff2B, nfG, nfB,
                pi1T, pi1B, pi2W, pi2B, v1T, v1B, v2W, v2B,
                ur1T, ur1B, ur2W, ur2B, r1T, r1B, r2W, r2B,
                pol_ref, rank_ref, val_ref, ur_ref):
    relu = jax.nn.relu
    xv = x_ref[0]
    h1 = relu(_mm(xv, w1T[...]) + b1[...])
    cell = _mm(h1, w2T[...]) + b2[...]
    cand = _mm(Rm[...], cell) + temb[...]
    sf = semb[...]
    for l in range(L):
        msgs = _mm(cand, vcT[l]) + vcB[l]
        su = _mm(A9[...], msgs)
        sf = _ln(sf + su, nsG[l], nsB[l])
        gath = _mm(AT3[...], sf)
        cu = _mm(gath, vsT[l]) + vsB[l]
        cand = _ln(cand + cu, ncG[l], ncB[l])
        hff = relu(_mm(cand, ff1T[l]) + ff1B[l])
        ffo = _mm(hff, ff2T[l]) + ff2B[l]
        cand = _ln(cand + ffo, nfG[l], nfB[l])
    hpi = relu(_mm(cand, pi1T[...]) + pi1B[...])
    pol_ref[0] = _mm_nt(pi2W[...], hpi) + pi2B[...]
    hr = relu(_mm(cand, r1T[...]) + r1B[...])
    rank_ref[0] = jax.nn.sigmoid(_mm_nt(r2W[...], hr) + r2B[...])
    gmax = jnp.max(cand, axis=0, keepdims=True)
    hv = relu(_mm(gmax, v1T[...]) + v1B[...])
    val = jnp.tanh(jnp.sum(hv * v2W[...], axis=1, keepdims=True) + v2B[...])
    val_ref[0] = jnp.broadcast_to(val, (1, 128))
    hu = relu(_mm(gmax, ur1T[...]) + ur1B[...])
    urv = jax.nn.sigmoid(jnp.sum(hu * ur2W[...], axis=1, keepdims=True)
                         + ur2B[...])
    ur_ref[0] = jnp.broadcast_to(urv, (1, 128))


def kernel(x, params):
    p = params
    B = x.shape[0]
    f32 = jnp.float32
    x2 = x.reshape(B, C_IN, 81).transpose(0, 2, 1)
    lys = p['layers']

    def stk(name, transpose=False):
        if transpose:
            return jnp.stack([lp[name].T for lp in lys])
        return jnp.stack([lp[name][None, :] for lp in lys])

    operands = [
        x2,
        p['conv1_w'].T, p['conv1_b'][None, :],
        p['conv2_w'].T, p['conv2_b'][None, :],
        jnp.asarray(_R_NP), jnp.tile(p['digit_embed'], (81, 1)),
        p['set_embed'], jnp.asarray(_A9_NP), jnp.asarray(_AT3_NP),
        stk('vc_w', True), stk('vc_b'),
        stk('vs_w', True), stk('vs_b'),
        stk('ns_g'), stk('ns_b'),
        stk('nc_g'), stk('nc_b'),
        stk('ff1_w', True), stk('ff1_b'),
        stk('ff2_w', True), stk('ff2_b'),
        stk('nf_g'), stk('nf_b'),
        p['pi1_w'].T, p['pi1_b'][None, :],
        p['pi2_w'], p['pi2_b'][:, None],
        p['v1_w'].T, p['v1_b'][None, :],
        p['v2_w'], p['v2_b'][:, None],
        p['ur1_w'].T, p['ur1_b'][None, :],
        p['ur2_w'], p['ur2_b'][:, None],
        p['r1_w'].T, p['r1_b'][None, :],
        p['r2_w'], p['r2_b'][:, None],
    ]

    def const_spec(arr):
        nd = arr.ndim
        return pl.BlockSpec(arr.shape, lambda i, _n=nd: (0,) * _n)

    in_specs = [pl.BlockSpec((1, 81, C_IN), lambda i: (i, 0, 0))]
    in_specs += [const_spec(a) for a in operands[1:]]

    out_shapes = [
        jax.ShapeDtypeStruct((B, 2, 729), f32),
        jax.ShapeDtypeStruct((B, 1, 729), f32),
        jax.ShapeDtypeStruct((B, 1, 128), f32),
        jax.ShapeDtypeStruct((B, 1, 128), f32),
    ]
    out_specs = [
        pl.BlockSpec((1, 2, 729), lambda i: (i, 0, 0)),
        pl.BlockSpec((1, 1, 729), lambda i: (i, 0, 0)),
        pl.BlockSpec((1, 1, 128), lambda i: (i, 0, 0)),
        pl.BlockSpec((1, 1, 128), lambda i: (i, 0, 0)),
    ]

    pol, rank, val, ur = pl.pallas_call(
        _fwd_kernel,
        grid=(B,),
        in_specs=in_specs,
        out_specs=out_specs,
        out_shape=out_shapes,
        compiler_params=pltpu.CompilerParams(
            dimension_semantics=("parallel",),
        ),
    )(*operands)

    policy = pol.reshape(B, 1458)
    return (policy, val[:, 0, 0], ur[:, 0, 0], rank.reshape(B, 729))


# speedup vs baseline: 1.9377x; 1.9377x over previous
import numpy as np

import jax
import jax.numpy as jnp
from jax.experimental import pallas as pl
from jax.experimental.pallas import tpu as pltpu

D = 128
L = 4
C_IN = 22


def _build_static():
    a = np.zeros((729, 3), dtype=np.int32)
    for cell in range(81):
        r = cell // 9
        c = cell % 9
        b = (r // 3) * 3 + c // 3
        for d in range(9):
            ci = cell * 9 + d
            a[ci, 0] = r * 9 + d
            a[ci, 1] = 81 + c * 9 + d
            a[ci, 2] = 162 + b * 9 + d
    A = np.zeros((243, 729), dtype=np.float32)
    for ci in range(729):
        for k in range(3):
            A[a[ci, k], ci] = 1.0
    R = np.zeros((729, 81), dtype=np.float32)
    for cell in range(81):
        for d in range(9):
            R[cell * 9 + d, cell] = 1.0
    return A, A.T.copy(), R


_A_NP, _AT_NP, _R_NP = _build_static()

_BF = jnp.bfloat16
_F32 = jnp.float32


def _dot(a, b):
    return jax.lax.dot_general(a, b, (((1,), (0,)), ((), ())),
                               preferred_element_type=_F32)


def _dot_nt(a, b):
    return jax.lax.dot_general(a, b, (((1,), (1,)), ((), ())),
                               preferred_element_type=_F32)


def _split(x):
    hi = x.astype(_BF)
    lo = (x - hi.astype(_F32)).astype(_BF)
    return hi, lo


def _mm3(x, w_hi, w_lo):
    xh, xl = _split(x)
    return _dot(xh, w_hi) + (_dot(xh, w_lo) + _dot(xl, w_hi))


def _mm3_nt(w_hi, w_lo, x):
    xh, xl = _split(x)
    return _dot_nt(w_hi, xh) + (_dot_nt(w_lo, xh) + _dot_nt(w_hi, xl))


def _ln(x, g, b):
    m = jnp.mean(x, axis=-1, keepdims=True)
    v = jnp.mean((x - m) ** 2, axis=-1, keepdims=True)
    return (x - m) / jnp.sqrt(v + 1e-5) * g + b


def _fwd_kernel(x_ref, w1h, w1l, b1, w2h, w2l, b2, Rm, temb, semb, Am, ATm,
                vcH, vcL, vcB, vsH, vsL, vsB, nsG, nsB, ncG, ncB,
                ff1H, ff1L, ff1B, ff2H, ff2L, ff2B, nfG, nfB,
                pi1H, pi1L, pi1B, pi2H, pi2L, pi2B,
                v1H, v1L, v1B, v2W, v2B,
                ur1H, ur1L, ur1B, ur2W, ur2B,
                r1H, r1L, r1B, r2H, r2L, r2B,
                pol_ref, rank_ref, val_ref, ur_ref):
    relu = jax.nn.relu
    xv = x_ref[0]
    h1 = relu(_mm3(xv, w1h[...], w1l[...]) + b1[...])
    cell = _mm3(h1, w2h[...], w2l[...]) + b2[...]
    ch, cl = _split(cell)
    cand = _dot(Rm[...], ch) + _dot(Rm[...], cl) + temb[...]
    sf = semb[...]
    for l in range(L):
        msgs = _mm3(cand, vcH[l], vcL[l]) + vcB[l]
        su = _dot(Am[...], msgs.astype(_BF)) * (1.0 / 9.0)
        sf = _ln(sf + su, nsG[l], nsB[l])
        gath = _dot(ATm[...], sf.astype(_BF)) * (1.0 / 3.0)
        cu = _mm3(gath, vsH[l], vsL[l]) + vsB[l]
        cand = _ln(cand + cu, ncG[l], ncB[l])
        hff = relu(_mm3(cand, ff1H[l], ff1L[l]) + ff1B[l])
        ffo = _mm3(hff, ff2H[l], ff2L[l]) + ff2B[l]
        cand = _ln(cand + ffo, nfG[l], nfB[l])
    hpi = relu(_mm3(cand, pi1H[...], pi1L[...]) + pi1B[...])
    pol_ref[0] = _mm3_nt(pi2H[...], pi2L[...], hpi) + pi2B[...]
    hr = relu(_mm3(cand, r1H[...], r1L[...]) + r1B[...])
    rank_ref[0] = jax.nn.sigmoid(_mm3_nt(r2H[...], r2L[...], hr) + r2B[...])
    gmax = jnp.max(cand, axis=0, keepdims=True)
    hv = relu(_mm3(gmax, v1H[...], v1L[...]) + v1B[...])
    val = jnp.tanh(jnp.sum(hv * v2W[...], axis=1, keepdims=True) + v2B[...])
    val_ref[0] = jnp.broadcast_to(val, (1, 128))
    hu = relu(_mm3(gmax, ur1H[...], ur1L[...]) + ur1B[...])
    urv = jax.nn.sigmoid(jnp.sum(hu * ur2W[...], axis=1, keepdims=True)
                         + ur2B[...])
    ur_ref[0] = jnp.broadcast_to(urv, (1, 128))


def _wsplit(w):
    hi = w.astype(_BF)
    lo = (w - hi.astype(_F32)).astype(_BF)
    return hi, lo


def kernel(x, params):
    p = params
    B = x.shape[0]
    x2 = x.reshape(B, C_IN, 81).transpose(0, 2, 1)
    lys = p['layers']

    def stk2(name):
        w = jnp.stack([lp[name].T for lp in lys])
        return _wsplit(w)

    def stkb(name):
        return jnp.stack([lp[name][None, :] for lp in lys])

    w1h, w1l = _wsplit(p['conv1_w'].T)
    w2h, w2l = _wsplit(p['conv2_w'].T)
    vcH, vcL = stk2('vc_w')
    vsH, vsL = stk2('vs_w')
    ff1H, ff1L = stk2('ff1_w')
    ff2H, ff2L = stk2('ff2_w')
    pi1H, pi1L = _wsplit(p['pi1_w'].T)
    pi2H, pi2L = _wsplit(p['pi2_w'])
    v1H, v1L = _wsplit(p['v1_w'].T)
    ur1H, ur1L = _wsplit(p['ur1_w'].T)
    r1H, r1L = _wsplit(p['r1_w'].T)
    r2H, r2L = _wsplit(p['r2_w'])

    operands = [
        x2,
        w1h, w1l, p['conv1_b'][None, :],
        w2h, w2l, p['conv2_b'][None, :],
        jnp.asarray(_R_NP, _BF), jnp.tile(p['digit_embed'], (81, 1)),
        p['set_embed'],
        jnp.asarray(_A_NP, _BF), jnp.asarray(_AT_NP, _BF),
        vcH, vcL, stkb('vc_b'),
        vsH, vsL, stkb('vs_b'),
        stkb('ns_g'), stkb('ns_b'),
        stkb('nc_g'), stkb('nc_b'),
        ff1H, ff1L, stkb('ff1_b'),
        ff2H, ff2L, stkb('ff2_b'),
        stkb('nf_g'), stkb('nf_b'),
        pi1H, pi1L, p['pi1_b'][None, :],
        pi2H, pi2L, p['pi2_b'][:, None],
        v1H, v1L, p['v1_b'][None, :],
        p['v2_w'], p['v2_b'][:, None],
        ur1H, ur1L, p['ur1_b'][None, :],
        p['ur2_w'], p['ur2_b'][:, None],
        r1H, r1L, p['r1_b'][None, :],
        r2H, r2L, p['r2_b'][:, None],
    ]

    def const_spec(arr):
        nd = arr.ndim
        return pl.BlockSpec(arr.shape, lambda i, _n=nd: (0,) * _n)

    in_specs = [pl.BlockSpec((1, 81, C_IN), lambda i: (i, 0, 0))]
    in_specs += [const_spec(a) for a in operands[1:]]

    out_shapes = [
        jax.ShapeDtypeStruct((B, 2, 729), _F32),
        jax.ShapeDtypeStruct((B, 1, 729), _F32),
        jax.ShapeDtypeStruct((B, 1, 128), _F32),
        jax.ShapeDtypeStruct((B, 1, 128), _F32),
    ]
    out_specs = [
        pl.BlockSpec((1, 2, 729), lambda i: (i, 0, 0)),
        pl.BlockSpec((1, 1, 729), lambda i: (i, 0, 0)),
        pl.BlockSpec((1, 1, 128), lambda i: (i, 0, 0)),
        pl.BlockSpec((1, 1, 128), lambda i: (i, 0, 0)),
    ]

    pol, rank, val, ur = pl.pallas_call(
        _fwd_kernel,
        grid=(B,),
        in_specs=in_specs,
        out_specs=out_specs,
        out_shape=out_shapes,
        compiler_params=pltpu.CompilerParams(
            dimension_semantics=("parallel",),
        ),
    )(*operands)

    policy = pol.reshape(B, 1458)
    return (policy, val[:, 0, 0], ur[:, 0, 0], rank.reshape(B, 729))


# speedup vs baseline: 3.8071x; 1.9647x over previous
import jax
import jax.numpy as jnp
from jax.experimental import pallas as pl
from jax.experimental.pallas import tpu as pltpu

D = 128
L = 4
C_IN = 22

_BF = jnp.bfloat16
_F32 = jnp.float32


def _dot(a, b):
    return jax.lax.dot_general(a, b, (((1,), (0,)), ((), ())),
                               preferred_element_type=_F32)


def _dot_nt(a, b):
    return jax.lax.dot_general(a, b, (((1,), (1,)), ((), ())),
                               preferred_element_type=_F32)


def _mmd(x, w_bf):
    return _dot(x.astype(_BF), w_bf)


def _mmd_nt(w_bf, x):
    return _dot_nt(w_bf, x.astype(_BF))


def _ln(x, g, b):
    m = jnp.mean(x, axis=-1, keepdims=True)
    v = jnp.mean((x - m) ** 2, axis=-1, keepdims=True)
    return (x - m) / jnp.sqrt(v + 1e-5) * g + b


def _fwd_kernel(x_ref, w1T, b1, w2T, b2, demb, semb,
                vcT, vcB, vsT, vsB, nsG, nsB, ncG, ncB,
                ff1T, ff1B, ff2T, ff2B, nfG, nfB,
                pi1T, pi1B, pi2W, pi2B, v1T, v1B, v2W, v2B,
                ur1T, ur1B, ur2W, ur2B, r1T, r1B, r2W, r2B,
                pol_ref, rank_ref, val_ref, ur_ref, s729, s243):
    relu = jax.nn.relu
    xv = x_ref[0]
    h1 = relu(_mmd(xv, w1T[...]) + b1[...])
    cell = _mmd(h1, w2T[...]) + b2[...]
    emb = demb[...]
    for i in range(81):
        s729[9 * i:9 * i + 9] = jnp.broadcast_to(cell[i:i + 1], (9, D)) + emb
    cand = s729[...]
    sf = semb[...]
    for l in range(L):
        msgs = _mmd(cand, vcT[l]) + vcB[l]
        s729[...] = msgs
        for r in range(9):
            acc = s729[81 * r:81 * r + 9]
            for c in range(1, 9):
                acc = acc + s729[81 * r + 9 * c:81 * r + 9 * c + 9]
            s243[9 * r:9 * r + 9] = acc
        for c in range(9):
            acc = s729[9 * c:9 * c + 9]
            for r in range(1, 9):
                acc = acc + s729[81 * r + 9 * c:81 * r + 9 * c + 9]
            s243[81 + 9 * c:81 + 9 * c + 9] = acc
        for bb in range(9):
            Rb, Cb = divmod(bb, 3)
            acc = None
            for rr in range(3):
                for cc in range(3):
                    o = 81 * (3 * Rb + rr) + 9 * (3 * Cb + cc)
                    blk = s729[o:o + 9]
                    acc = blk if acc is None else acc + blk
            s243[162 + 9 * bb:162 + 9 * bb + 9] = acc
        sf = _ln(sf + s243[...] / 9.0, nsG[l], nsB[l])
        s243[...] = sf
        for r in range(9):
            sfr = s243[9 * r:9 * r + 9]
            for c in range(9):
                bb = (r // 3) * 3 + c // 3
                blk = (sfr + s243[81 + 9 * c:81 + 9 * c + 9]) \
                    + s243[162 + 9 * bb:162 + 9 * bb + 9]
                s729[81 * r + 9 * c:81 * r + 9 * c + 9] = blk
        gath = s729[...] / 3.0
        cu = _mmd(gath, vsT[l]) + vsB[l]
        cand = _ln(cand + cu, ncG[l], ncB[l])
        hff = relu(_mmd(cand, ff1T[l]) + ff1B[l])
        ffo = _mmd(hff, ff2T[l]) + ff2B[l]
        cand = _ln(cand + ffo, nfG[l], nfB[l])
    hpi = relu(_mmd(cand, pi1T[...]) + pi1B[...])
    pol_ref[0] = _mmd_nt(pi2W[...], hpi) + pi2B[...]
    hr = relu(_mmd(cand, r1T[...]) + r1B[...])
    rank_ref[0] = jax.nn.sigmoid(_mmd_nt(r2W[...], hr) + r2B[...])
    gmax = jnp.max(cand, axis=0, keepdims=True)
    hv = relu(_mmd(gmax, v1T[...]) + v1B[...])
    val = jnp.tanh(_mmd_nt(v2W[...], hv) + v2B[...])
    val_ref[0] = jnp.broadcast_to(val, (1, 128))
    hu = relu(_mmd(gmax, ur1T[...]) + ur1B[...])
    urv = jax.nn.sigmoid(_mmd_nt(ur2W[...], hu) + ur2B[...])
    ur_ref[0] = jnp.broadcast_to(urv, (1, 128))


def kernel(x, params):
    p = params
    B = x.shape[0]
    x2 = x.reshape(B, C_IN, 81).transpose(0, 2, 1)
    lys = p['layers']

    def stkw(name):
        return jnp.stack([lp[name].T.astype(_BF) for lp in lys])

    def stkb(name):
        return jnp.stack([lp[name][None, :] for lp in lys])

    operands = [
        x2,
        p['conv1_w'].T.astype(_BF), p['conv1_b'][None, :],
        p['conv2_w'].T.astype(_BF), p['conv2_b'][None, :],
        p['digit_embed'], p['set_embed'],
        stkw('vc_w'), stkb('vc_b'),
        stkw('vs_w'), stkb('vs_b'),
        stkb('ns_g'), stkb('ns_b'),
        stkb('nc_g'), stkb('nc_b'),
        stkw('ff1_w'), stkb('ff1_b'),
        stkw('ff2_w'), stkb('ff2_b'),
        stkb('nf_g'), stkb('nf_b'),
        p['pi1_w'].T.astype(_BF), p['pi1_b'][None, :],
        p['pi2_w'].astype(_BF), p['pi2_b'][:, None],
        p['v1_w'].T.astype(_BF), p['v1_b'][None, :],
        p['v2_w'].astype(_BF), p['v2_b'][:, None],
        p['ur1_w'].T.astype(_BF), p['ur1_b'][None, :],
        p['ur2_w'].astype(_BF), p['ur2_b'][:, None],
        p['r1_w'].T.astype(_BF), p['r1_b'][None, :],
        p['r2_w'].astype(_BF), p['r2_b'][:, None],
    ]

    def const_spec(arr):
        nd = arr.ndim
        return pl.BlockSpec(arr.shape, lambda i, _n=nd: (0,) * _n)

    in_specs = [pl.BlockSpec((1, 81, C_IN), lambda i: (i, 0, 0))]
    in_specs += [const_spec(a) for a in operands[1:]]

    out_shapes = [
        jax.ShapeDtypeStruct((B, 2, 729), _F32),
        jax.ShapeDtypeStruct((B, 1, 729), _F32),
        jax.ShapeDtypeStruct((B, 1, 128), _F32),
        jax.ShapeDtypeStruct((B, 1, 128), _F32),
    ]
    out_specs = [
        pl.BlockSpec((1, 2, 729), lambda i: (i, 0, 0)),
        pl.BlockSpec((1, 1, 729), lambda i: (i, 0, 0)),
        pl.BlockSpec((1, 1, 128), lambda i: (i, 0, 0)),
        pl.BlockSpec((1, 1, 128), lambda i: (i, 0, 0)),
    ]

    pol, rank, val, ur = pl.pallas_call(
        _fwd_kernel,
        grid=(B,),
        in_specs=in_specs,
        out_specs=out_specs,
        out_shape=out_shapes,
        scratch_shapes=[
            pltpu.VMEM((729, D), _F32),
            pltpu.VMEM((243, D), _F32),
        ],
        compiler_params=pltpu.CompilerParams(
            dimension_semantics=("parallel",),
        ),
    )(*operands)

    policy = pol.reshape(B, 1458)
    return (policy, val[:, 0, 0], ur[:, 0, 0], rank.reshape(B, 729))
